# baseline (speedup 1.0000x reference)
import functools

import jax
import jax.numpy as jnp
import numpy as np
from jax import lax
from jax.experimental import pallas as pl
from jax.experimental.pallas import tpu as pltpu
from jax.experimental.pallas import tpu_sc as plsc

B = 16384
D = 32
H1 = 64
H2 = 32
NROWS = 1000000
QROWS = NROWS * D // 128

NC = 2
NS = 16
NW = NC * NS
BPW = B // NW
NG = BPW // 16


WBLK = 24576
SUB = WBLK // 128
RGRID = -(-NROWS // WBLK)
_SEL = np.zeros((128, 128), np.float32)
for _m in range(4):
  for _a in range(32):
    _SEL[32 * _m + _a, 4 * _a + _m] = 1.0


def _tc_relayout_body(x1, x2, x3, x4, a_ref, y1, y2, y3, y4):
  a = a_ref[...]
  nvalid = NROWS - pl.program_id(0) * WBLK
  lanes = lax.broadcasted_iota(jnp.int32, (D, WBLK), 1)
  validf = (lanes < nvalid).astype(jnp.float32)
  for xr, yr in ((x1, y1), (x2, y2), (x3, y3), (x4, y4)):
    x = xr[...] * validf
    for s in range(SUB):
      xt = x[:, 128 * s:128 * (s + 1)].T
      z = a @ xt
      for m in range(4):
        yr[32 * s:32 * (s + 1), 32 * m:32 * (m + 1)] = (
            z[32 * m:32 * (m + 1), :])


def _tc_relayout(t1, t2, t3, t4):
  sel = jnp.asarray(_SEL)
  inblk = pl.BlockSpec((D, WBLK), lambda i: (0, i))
  outblk = pl.BlockSpec((WBLK * D // 128, 128), lambda i: (i, 0))
  oshape = jax.ShapeDtypeStruct((QROWS, 128), jnp.float32)
  return pl.pallas_call(
      _tc_relayout_body,
      grid=(RGRID,),
      in_specs=[inblk, inblk, inblk, inblk,
                pl.BlockSpec((128, 128), lambda i: (0, 0))],
      out_specs=[outblk, outblk, outblk, outblk],
      out_shape=[oshape, oshape, oshape, oshape],
      compiler_params=pltpu.CompilerParams(
          vmem_limit_bytes=100 * 1024 * 1024),
  )(t1, t2, t3, t4, sel)


def _sc_gather(user_idx, item_idx, ug_lin, ig_lin, um_lin, im_lin):
  mesh = plsc.VectorSubcoreMesh(core_axis_name="c", subcore_axis_name="s")

  @functools.partial(
      pl.kernel,
      out_type=[jax.ShapeDtypeStruct((B, D), jnp.float32) for _ in range(4)],
      mesh=mesh,
      scratch_types=[
          pltpu.VMEM((BPW,), jnp.int32),
          pltpu.VMEM((BPW,), jnp.int32),
          pltpu.VMEM((BPW, D), jnp.float32),
          pltpu.VMEM((BPW, D), jnp.float32),
          pltpu.VMEM((BPW, D), jnp.float32),
          pltpu.VMEM((BPW, D), jnp.float32),
          pltpu.SemaphoreType.DMA,
      ],
      compiler_params=pltpu.CompilerParams(use_tc_tiling_on_sc=False),
  )
  def k(ui_hbm, ii_hbm, ug_hbm, ig_hbm, um_hbm, im_hbm,
        oug, oig, oum, oim,
        idx_u, idx_i, r_ug, r_ig, r_um, r_im, sem):
    wid = lax.axis_index("s") * NC + lax.axis_index("c")
    base = wid * BPW
    pltpu.sync_copy(ui_hbm.at[pl.ds(base, BPW)], idx_u)
    pltpu.sync_copy(ii_hbm.at[pl.ds(base, BPW)], idx_i)
    c1 = pltpu.async_copy(ug_hbm.at[idx_u], r_ug, sem)
    c2 = pltpu.async_copy(ig_hbm.at[idx_i], r_ig, sem)
    c3 = pltpu.async_copy(um_hbm.at[idx_u], r_um, sem)
    c4 = pltpu.async_copy(im_hbm.at[idx_i], r_im, sem)
    c1.wait()
    pltpu.sync_copy(r_ug, oug.at[pl.ds(base, BPW)])
    c2.wait()
    pltpu.sync_copy(r_ig, oig.at[pl.ds(base, BPW)])
    c3.wait()
    pltpu.sync_copy(r_um, oum.at[pl.ds(base, BPW)])
    c4.wait()
    pltpu.sync_copy(r_im, oim.at[pl.ds(base, BPW)])

  return k(user_idx, item_idx, ug_lin, ig_lin, um_lin, im_lin)


BLK = 2048


def _tc_mlp_body(ug, ig, um, im, w1t, b1, w2t, b2, wog, wom, bo, out):
  x = jnp.concatenate([um[...], im[...]], axis=1)
  h = jnp.maximum(x @ w1t[...] + b1[...], 0.0)
  h2 = jnp.maximum(h @ w2t[...] + b2[...], 0.0)
  g = jnp.sum(ug[...] * ig[...] * wog[...], axis=1)
  logit = g + jnp.squeeze(h2 @ wom[...], axis=-1) + bo[0, 0]
  out[...] = jax.nn.sigmoid(logit)


def _tc_mlp(ugr, igr, umr, imr, W1, b1, W2, b2, Wo, bo):
  wog = Wo[0, :D].reshape(1, D)
  wom = Wo[0, D:].reshape(H2, 1)
  b1r = b1.reshape(1, H1)
  b2r = b2.reshape(1, H2)
  bor = bo.reshape(1, 1)

  grid = (B // BLK,)
  rowblk = lambda d: pl.BlockSpec((BLK, d), lambda i: (i, 0))
  rep = lambda s: pl.BlockSpec(s, lambda i: (0,) * len(s))
  return pl.pallas_call(
      _tc_mlp_body,
      grid=grid,
      in_specs=[
          rowblk(D), rowblk(D), rowblk(D), rowblk(D),
          rep((2 * D, H1)), rep((1, H1)),
          rep((H1, H2)), rep((1, H2)),
          rep((1, D)), rep((H2, 1)), rep((1, 1)),
      ],
      out_specs=pl.BlockSpec((BLK,), lambda i: (i,)),
      out_shape=jax.ShapeDtypeStruct((B,), jnp.float32),
  )(ugr, igr, umr, imr, W1.T, b1r, W2.T, b2r, wog, wom, bor)


def kernel(user_idx, item_idx, ue_gmf, ie_gmf, ue_mlp, ie_mlp,
           W1, b1, W2, b2, Wo, bo):
  ui = user_idx.astype(jnp.int32)
  ii = item_idx.astype(jnp.int32)
  ug_q, ig_q, um_q, im_q = _tc_relayout(
      ue_gmf.T, ie_gmf.T, ue_mlp.T, ie_mlp.T)
  lin = lambda t: t.reshape(NROWS, D)
  ugr, igr, umr, imr = _sc_gather(
      ui, ii, lin(ug_q), lin(ig_q), lin(um_q), lin(im_q))
  return _tc_mlp(ugr, igr, umr, imr, W1, b1, W2, b2, Wo, bo)


# speedup vs baseline: 1.0018x; 1.0018x over previous
import functools

import jax
import jax.numpy as jnp
import numpy as np
from jax import lax
from jax.experimental import pallas as pl
from jax.experimental.pallas import tpu as pltpu
from jax.experimental.pallas import tpu_sc as plsc

B = 16384
D = 32
H1 = 64
H2 = 32
NROWS = 1000000
QROWS = NROWS * D // 128

NC = 2
NS = 16
NW = NC * NS
BPW = B // NW


WBLK = 20480
SUB = WBLK // 128
RGRID = -(-NROWS // WBLK)
_SEL = np.zeros((128, 128), np.float32)
for _m in range(4):
  for _a in range(32):
    _SEL[32 * _m + _a, 4 * _a + _m] = 1.0


def _tc_relayout_body(x1, x2, x3, x4, a_ref, y1, y2, y3, y4):
  a = a_ref[...]
  nvalid = NROWS - pl.program_id(0) * WBLK
  lanes = lax.broadcasted_iota(jnp.int32, (D, WBLK), 1)
  validf = (lanes < nvalid).astype(jnp.float32)
  for xr, yr in ((x1, y1), (x2, y2), (x3, y3), (x4, y4)):
    x = xr[...] * validf
    for s in range(SUB):
      xt = x[:, 128 * s:128 * (s + 1)].T
      z = a @ xt
      for m in range(4):
        yr[32 * s:32 * (s + 1), 32 * m:32 * (m + 1)] = (
            z[32 * m:32 * (m + 1), :])


def _tc_relayout(t1, t2, t3, t4):
  sel = jnp.asarray(_SEL)
  inblk = pl.BlockSpec((D, WBLK), lambda i: (0, i))
  outblk = pl.BlockSpec((WBLK * D // 128, 128), lambda i: (i, 0))
  oshape = jax.ShapeDtypeStruct((QROWS, 128), jnp.float32)
  return pl.pallas_call(
      _tc_relayout_body,
      grid=(RGRID,),
      in_specs=[inblk, inblk, inblk, inblk,
                pl.BlockSpec((128, 128), lambda i: (0, 0))],
      out_specs=[outblk, outblk, outblk, outblk],
      out_shape=[oshape, oshape, oshape, oshape],
  )(t1, t2, t3, t4, sel)


def _sc_gather(user_idx, item_idx, ug_lin, ig_lin, um_lin, im_lin):
  mesh = plsc.VectorSubcoreMesh(core_axis_name="c", subcore_axis_name="s")

  @functools.partial(
      pl.kernel,
      out_type=[jax.ShapeDtypeStruct((B, D), jnp.float32) for _ in range(4)],
      mesh=mesh,
      scratch_types=[
          pltpu.VMEM((BPW,), jnp.int32),
          pltpu.VMEM((BPW,), jnp.int32),
          pltpu.VMEM((BPW, D), jnp.float32),
          pltpu.VMEM((BPW, D), jnp.float32),
          pltpu.VMEM((BPW, D), jnp.float32),
          pltpu.VMEM((BPW, D), jnp.float32),
          pltpu.SemaphoreType.DMA,
      ],
      compiler_params=pltpu.CompilerParams(use_tc_tiling_on_sc=False),
  )
  def k(ui_hbm, ii_hbm, ug_hbm, ig_hbm, um_hbm, im_hbm,
        oug, oig, oum, oim,
        idx_u, idx_i, r_ug, r_ig, r_um, r_im, sem):
    wid = lax.axis_index("s") * NC + lax.axis_index("c")
    base = wid * BPW
    pltpu.sync_copy(ui_hbm.at[pl.ds(base, BPW)], idx_u)
    pltpu.sync_copy(ii_hbm.at[pl.ds(base, BPW)], idx_i)
    c1 = pltpu.async_copy(ug_hbm.at[idx_u], r_ug, sem)
    c2 = pltpu.async_copy(ig_hbm.at[idx_i], r_ig, sem)
    c3 = pltpu.async_copy(um_hbm.at[idx_u], r_um, sem)
    c4 = pltpu.async_copy(im_hbm.at[idx_i], r_im, sem)
    c1.wait()
    pltpu.sync_copy(r_ug, oug.at[pl.ds(base, BPW)])
    c2.wait()
    pltpu.sync_copy(r_ig, oig.at[pl.ds(base, BPW)])
    c3.wait()
    pltpu.sync_copy(r_um, oum.at[pl.ds(base, BPW)])
    c4.wait()
    pltpu.sync_copy(r_im, oim.at[pl.ds(base, BPW)])

  return k(user_idx, item_idx, ug_lin, ig_lin, um_lin, im_lin)


BLK = 2048


def _tc_mlp_body(ug, ig, um, im, w1t, b1, w2t, b2, wog, wom, bo, out):
  x = jnp.concatenate([um[...], im[...]], axis=1)
  h = jnp.maximum(x @ w1t[...] + b1[...], 0.0)
  h2 = jnp.maximum(h @ w2t[...] + b2[...], 0.0)
  g = jnp.sum(ug[...] * ig[...] * wog[...], axis=1)
  logit = g + jnp.squeeze(h2 @ wom[...], axis=-1) + bo[0, 0]
  out[...] = jax.nn.sigmoid(logit)


def _tc_mlp(ugr, igr, umr, imr, W1, b1, W2, b2, Wo, bo):
  wog = Wo[0, :D].reshape(1, D)
  wom = Wo[0, D:].reshape(H2, 1)
  b1r = b1.reshape(1, H1)
  b2r = b2.reshape(1, H2)
  bor = bo.reshape(1, 1)

  grid = (B // BLK,)
  rowblk = lambda d: pl.BlockSpec((BLK, d), lambda i: (i, 0))
  rep = lambda s: pl.BlockSpec(s, lambda i: (0,) * len(s))
  return pl.pallas_call(
      _tc_mlp_body,
      grid=grid,
      in_specs=[
          rowblk(D), rowblk(D), rowblk(D), rowblk(D),
          rep((2 * D, H1)), rep((1, H1)),
          rep((H1, H2)), rep((1, H2)),
          rep((1, D)), rep((H2, 1)), rep((1, 1)),
      ],
      out_specs=pl.BlockSpec((BLK,), lambda i: (i,)),
      out_shape=jax.ShapeDtypeStruct((B,), jnp.float32),
  )(ugr, igr, umr, imr, W1.T, b1r, W2.T, b2r, wog, wom, bor)


def kernel(user_idx, item_idx, ue_gmf, ie_gmf, ue_mlp, ie_mlp,
           W1, b1, W2, b2, Wo, bo):
  ui = user_idx.astype(jnp.int32)
  ii = item_idx.astype(jnp.int32)
  ug_q, ig_q, um_q, im_q = _tc_relayout(
      ue_gmf.T, ie_gmf.T, ue_mlp.T, ie_mlp.T)
  lin = lambda t: t.reshape(NROWS, D)
  ugr, igr, umr, imr = _sc_gather(
      ui, ii, lin(ug_q), lin(ig_q), lin(um_q), lin(im_q))
  return _tc_mlp(ugr, igr, umr, imr, W1, b1, W2, b2, Wo, bo)
